# baseline (speedup 1.0000x reference)
import functools

import jax
import jax.numpy as jnp
from jax import lax
from jax.experimental import pallas as pl
from jax.experimental.pallas import tpu as pltpu
from jax.experimental.pallas import tpu_sc as plsc

_NC = 2
_NS = 16

_B, _H, _W, _C = 8, 56, 56, 192
_HW = _H * _W
_ROWS = _B * _HW
_NW = _NC * _NS
_RPW = _ROWS // _NW
_SEG_PER_BATCH = _HW // _RPW
_TPW = _RPW // 8
_GSZ = 7
_NG = _TPW // _GSZ
_NRING = 3


def _shuffle_body(x_hbm, r_hbm, out_hbm, idx_v, bufs, rsem, osem):
    wid = lax.axis_index("s") * _NC + lax.axis_index("c")
    batch = wid // _SEG_PER_BATCH
    seg = wid % _SEG_PER_BATCH
    out_base = wid * _RPW
    boff = (batch * _HW).astype(jnp.int32)

    pltpu.sync_copy(r_hbm.at[pl.ds(seg * _RPW, _RPW)], idx_v.at[pl.ds(0, _RPW)])

    def read_tile(t, g, b):
        v = idx_v[pl.ds(t * 8, 16)] + boff
        for j in range(8):
            pltpu.async_copy(
                x_hbm.at[pl.ds(v[j], 1), :],
                bufs.at[g, b].at[pl.ds(j, 1), :],
                rsem.at[g],
            )

    def read_group(k):
        g = k % _NRING
        for b in range(_GSZ):
            read_tile(k * _GSZ + b, g, b)

    def drain_reads(g, b):
        pltpu.make_async_copy(
            x_hbm.at[pl.ds(0, 8), :], bufs.at[g, b], rsem.at[g]
        ).wait()

    def out_copy(k, b):
        t = k * _GSZ + b
        return pltpu.make_async_copy(
            bufs.at[k % _NRING, b],
            out_hbm.at[pl.ds(out_base + t * 8, 8), :],
            osem.at[k % _NRING],
        )

    read_group(0)
    read_group(1)
    for k in range(_NG):
        if k >= 2:
            for b in range(_GSZ):
                out_copy(k - 2, b).wait()
        if 1 <= k <= _NG - 2:
            read_group(k + 1)
        for b in range(_GSZ):
            drain_reads(k % _NRING, b)
        for b in range(_GSZ):
            out_copy(k, b).start()
    for k in (_NG - 2, _NG - 1):
        for b in range(_GSZ):
            out_copy(k, b).wait()


@jax.jit
def kernel(x, r):
    B, H, W, C = x.shape
    xf = x.reshape(B * H * W, C)
    r32 = r.astype(jnp.int32)

    mesh = plsc.VectorSubcoreMesh(
        core_axis_name="c", subcore_axis_name="s",
        num_cores=_NC, num_subcores=_NS,
    )
    run = pl.kernel(
        _shuffle_body,
        out_type=jax.ShapeDtypeStruct((B * H * W, C), x.dtype),
        mesh=mesh,
        scratch_types=[
            pltpu.VMEM((_RPW + 16,), jnp.int32),
            pltpu.VMEM((_NRING, _GSZ, 8, _C), jnp.float32),
            pltpu.SemaphoreType.DMA((_NRING,)),
            pltpu.SemaphoreType.DMA((_NRING,)),
        ],
    )
    out = run(xf, r32)
    return out.reshape(B, H, W, C)


# speedup vs baseline: 1.1677x; 1.1677x over previous
import functools

import jax
import jax.numpy as jnp
from jax import lax
from jax.experimental import pallas as pl
from jax.experimental.pallas import tpu as pltpu
from jax.experimental.pallas import tpu_sc as plsc

_NC = 2
_NS = 16

_B, _H, _W, _C = 8, 56, 56, 192
_HW = _H * _W
_ROWS = _B * _HW
_NW = _NC * _NS
_RPW = _ROWS // _NW
_SEG_PER_BATCH = _HW // _RPW
_TPW = _RPW // 8
_D = 7
_G = _TPW // _D


def _shuffle_body(x_hbm, r_hbm, out_hbm, idx_v, bufs, rsem, osem):
    wid = lax.axis_index("s") * _NC + lax.axis_index("c")
    batch = wid // _SEG_PER_BATCH
    seg = wid % _SEG_PER_BATCH
    out_base = wid * _RPW
    boff = (batch * _HW).astype(jnp.int32)

    pltpu.sync_copy(r_hbm.at[pl.ds(seg * _RPW, _RPW)], idx_v.at[pl.ds(0, _RPW)])

    def read_tile(t, b):
        v = idx_v[pl.ds(t * 8, 16)] + boff
        for j in range(8):
            sg = v[j]
            pltpu.async_copy(
                x_hbm.at[pl.ds(sg, 1), :],
                bufs.at[b].at[pl.ds(j, 1), :],
                rsem.at[b],
            )

    def drain_reads(b):
        pltpu.make_async_copy(
            x_hbm.at[pl.ds(0, 8), :], bufs.at[b], rsem.at[b]
        ).wait()

    def out_copy(t, b):
        return pltpu.make_async_copy(
            bufs.at[b],
            out_hbm.at[pl.ds(out_base + t * 8, 8), :],
            osem.at[b],
        )

    for b in range(_D):
        read_tile(b, b)

    def gen(g, _):
        for b in range(_D):
            t = g * _D + b
            drain_reads(b)
            out_copy(t, b).start()
        for b in range(_D):
            t = g * _D + b
            out_copy(t, b).wait()
            read_tile(t + _D, b)
        return _

    lax.fori_loop(0, _G - 1, gen, None)

    for b in range(_D):
        t = (_G - 1) * _D + b
        drain_reads(b)
        out_copy(t, b).start()
    for b in range(_D):
        t = (_G - 1) * _D + b
        out_copy(t, b).wait()


@jax.jit
def kernel(x, r):
    B, H, W, C = x.shape
    xf = x.reshape(B * H * W, C)
    r32 = r.astype(jnp.int32)

    mesh = plsc.VectorSubcoreMesh(
        core_axis_name="c", subcore_axis_name="s",
        num_cores=_NC, num_subcores=_NS,
    )
    run = pl.kernel(
        _shuffle_body,
        out_type=jax.ShapeDtypeStruct((B * H * W, C), x.dtype),
        mesh=mesh,
        scratch_types=[
            pltpu.VMEM((_RPW + 16,), jnp.int32),
            pltpu.VMEM((_D, 8, _C), jnp.float32),
            pltpu.SemaphoreType.DMA((_D,)),
            pltpu.SemaphoreType.DMA((_D,)),
        ],
    )
    out = run(xf, r32)
    return out.reshape(B, H, W, C)
